# baseline (speedup 1.0000x reference)
import functools

import jax
import jax.numpy as jnp
from jax import lax
from jax.experimental import pallas as pl
from jax.experimental.pallas import tpu as pltpu
from jax.experimental.pallas import tpu_sc as plsc

_VOCAB = 38462 * 26
_D = 16
_BATCH = 16384
_NF = 26
_B = _BATCH * _NF

_NC = 2
_NS = 16
_NW = _NC * _NS
_BPW = _B // _NW
_CHUNK = 1664
_NCHUNK = _BPW // _CHUNK


def _make_gather():
    mesh = plsc.VectorSubcoreMesh(core_axis_name="c", subcore_axis_name="s")

    @functools.partial(
        pl.kernel,
        mesh=mesh,
        out_type=jax.ShapeDtypeStruct((_B, _D), jnp.float32),
        scratch_types=[
            pltpu.VMEM((_CHUNK,), jnp.int32),
            pltpu.VMEM((_CHUNK, _D), jnp.float32),
            pltpu.SemaphoreType.DMA,
        ],
        compiler_params=pltpu.CompilerParams(use_tc_tiling_on_sc=False),
    )
    def gather_kernel(idx_hbm, table_hbm, out_hbm, idx_v, rows_v, sem):
        wid = lax.axis_index("s") * _NC + lax.axis_index("c")
        wbase = wid * _BPW

        def body(j, carry):
            base = wbase + j * _CHUNK
            pltpu.sync_copy(idx_hbm.at[pl.ds(base, _CHUNK)], idx_v)
            pltpu.async_copy(table_hbm.at[idx_v], rows_v, sem).wait()
            pltpu.sync_copy(rows_v, out_hbm.at[pl.ds(base, _CHUNK)])
            return carry

        lax.fori_loop(0, _NCHUNK, body, 0)

    return gather_kernel


_gather = _make_gather()


def kernel(x, table):
    idx = x.reshape(_B).astype(jnp.int32)
    out = _gather(idx, table)
    return out.reshape(_BATCH, _NF, _D)


# speedup vs baseline: 1.0107x; 1.0107x over previous
import functools

import jax
import jax.numpy as jnp
from jax import lax
from jax.experimental import pallas as pl
from jax.experimental.pallas import tpu as pltpu
from jax.experimental.pallas import tpu_sc as plsc

_VOCAB = 38462 * 26
_D = 16
_BATCH = 16384
_NF = 26
_B = _BATCH * _NF

_NC = 2
_NS = 16
_NW = _NC * _NS
_BPW = _B // _NW
_CHUNK = 1664
_NCHUNK = _BPW // _CHUNK


def _make_gather():
    mesh = plsc.VectorSubcoreMesh(core_axis_name="c", subcore_axis_name="s")

    @functools.partial(
        pl.kernel,
        mesh=mesh,
        out_type=jax.ShapeDtypeStruct((_B, _D), jnp.float32),
        scratch_types=[
            pltpu.VMEM((_BPW,), jnp.int32),
            pltpu.VMEM((2, _CHUNK, _D), jnp.float32),
            pltpu.SemaphoreType.DMA,
            pltpu.SemaphoreType.DMA,
            pltpu.SemaphoreType.DMA,
            pltpu.SemaphoreType.DMA,
        ],
        compiler_params=pltpu.CompilerParams(use_tc_tiling_on_sc=False),
    )
    def gather_kernel(idx_hbm, table_hbm, out_hbm, idx_v, rows_v,
                      sem_g0, sem_g1, sem_s0, sem_s1):
        wid = lax.axis_index("s") * _NC + lax.axis_index("c")
        wbase = wid * _BPW

        sem_g = (sem_g0, sem_g1)
        sem_s = (sem_s0, sem_s1)

        pltpu.sync_copy(idx_hbm.at[pl.ds(wbase, _BPW)], idx_v)

        def gather(j):
            b = j % 2
            return pltpu.async_copy(
                table_hbm.at[idx_v.at[pl.ds(j * _CHUNK, _CHUNK)]],
                rows_v.at[b], sem_g[b])

        def store(j):
            b = j % 2
            return pltpu.async_copy(
                rows_v.at[b], out_hbm.at[pl.ds(wbase + j * _CHUNK, _CHUNK)],
                sem_s[b])

        g = [None] * _NCHUNK
        s = [None] * _NCHUNK
        g[0] = gather(0)
        for j in range(_NCHUNK):
            if j + 1 < _NCHUNK:
                if j >= 1:
                    s[j - 1].wait()
                g[j + 1] = gather(j + 1)
            g[j].wait()
            s[j] = store(j)
        s[_NCHUNK - 1].wait()
        if _NCHUNK >= 2:
            s[_NCHUNK - 2].wait()

    return gather_kernel


_gather = _make_gather()


def kernel(x, table):
    idx = x.reshape(_B).astype(jnp.int32)
    out = _gather(idx, table)
    return out.reshape(_BATCH, _NF, _D)
